# baseline (speedup 1.0000x reference)
import functools

import jax
import jax.numpy as jnp
import numpy as np
from jax.experimental import pallas as pl
from jax.experimental.pallas import tpu as pltpu

_TRANS_WEIGHT = 1.0
_ROT_WEIGHT = 100.0
_PI = np.float32(np.pi)
_TWO_PI = np.float32(2.0 * np.pi)
_W = 128


def _wrap_correction(a, hi, lo):
    c = jnp.where(a > hi, jnp.float32(-_TWO_PI), jnp.float32(0.0))
    return jnp.where(a < lo, jnp.float32(_TWO_PI), c)


def _loss_kernel(const_ref, p_ref, t_ref, out_ref, acc_ref, *, n_steps, bb,
                 inv_n):
    j = pl.program_id(0)

    his = [const_ref[ph * 24 + 0:ph * 24 + 8, :] for ph in range(3)]
    los = [const_ref[ph * 24 + 8:ph * 24 + 16, :] for ph in range(3)]

    accs = [jnp.zeros((8, _W), jnp.float32) for _ in range(3)]
    for i in range(bb // 8):
        ph = (8 * i) % 3
        p = p_ref[i * 8:(i + 1) * 8, :]
        t = t_ref[i * 8:(i + 1) * 8, :]
        d = (p - t) + (_wrap_correction(p, his[ph], los[ph])
                       - _wrap_correction(t, his[ph], los[ph]))
        accs[ph] = accs[ph] + d * d

    for ph in range(3):
        @pl.when(j == 0)
        def _(ph=ph):
            acc_ref[ph * 8:(ph + 1) * 8, :] = accs[ph]

        @pl.when(j > 0)
        def _(ph=ph):
            acc_ref[ph * 8:(ph + 1) * 8, :] += accs[ph]

    @pl.when(j == n_steps - 1)
    def _():
        s_trans = jnp.float32(0.0)
        s_all = jnp.float32(0.0)
        for ph in range(3):
            a = acc_ref[ph * 8:(ph + 1) * 8, :]
            mask = const_ref[ph * 24 + 16:ph * 24 + 24, :]
            s_trans = s_trans + jnp.sum(a * mask)
            s_all = s_all + jnp.sum(a)
        trans_loss = s_trans * inv_n * _TRANS_WEIGHT
        rot_loss = (s_all - s_trans) * inv_n * _ROT_WEIGHT
        out_ref[0, 0] = trans_loss + rot_loss
        out_ref[0, 1] = trans_loss
        out_ref[0, 2] = rot_loss


def kernel(pred, target, *, interpret=False):
    B, T, D = pred.shape
    N = B * T * D
    R = N // _W
    BB = 6144
    G = R // BB
    p2 = pred.reshape(R, _W)
    t2 = target.reshape(R, _W)

    tiles = []
    s = np.arange(8)[:, None]
    c = np.arange(_W)[None, :]
    for ph in range(3):
        d_idx = (2 * (ph + s) + c) % D
        is_rot = d_idx >= 3
        tiles.append(np.where(is_rot, _PI, np.inf).astype(np.float32))
        tiles.append(np.where(is_rot, -_PI, -np.inf).astype(np.float32))
        tiles.append((~is_rot).astype(np.float32))
    const = jnp.asarray(np.concatenate(tiles))

    n_per_half = N // 2
    out = pl.pallas_call(
        functools.partial(
            _loss_kernel, n_steps=G, bb=BB, inv_n=np.float32(1.0 / n_per_half)
        ),
        grid=(G,),
        in_specs=[
            pl.BlockSpec((72, _W), lambda j: (0, 0)),
            pl.BlockSpec((BB, _W), lambda j: (j, 0)),
            pl.BlockSpec((BB, _W), lambda j: (j, 0)),
        ],
        out_specs=pl.BlockSpec(memory_space=pltpu.SMEM),
        out_shape=jax.ShapeDtypeStruct((1, 3), jnp.float32),
        scratch_shapes=[pltpu.VMEM((24, _W), jnp.float32)],
        compiler_params=pltpu.CompilerParams(
            dimension_semantics=("arbitrary",),
        ),
        name="weighted_mse_loss",
        interpret=interpret,
    )(const, p2, t2)

    return (out[0, 0], out[0, 1], out[0, 2])


# speedup vs baseline: 53.0804x; 53.0804x over previous
import functools

import jax
import jax.numpy as jnp
import numpy as np
from jax.experimental import pallas as pl
from jax.experimental.pallas import tpu as pltpu

_TRANS_WEIGHT = 1.0
_ROT_WEIGHT = 100.0
_PI = np.float32(np.pi)
_TWO_PI = np.float32(2.0 * np.pi)


def _wrap_correction(a):
    c = jnp.where(a > _PI, jnp.float32(-_TWO_PI), jnp.float32(0.0))
    return jnp.where(a < -_PI, jnp.float32(_TWO_PI), c)


def _loss_kernel(p_ref, t_ref, out_ref, acc_ref, *, n_steps, bbr, inv_n):
    j = pl.program_id(0)

    acc_t = jnp.zeros((8, 128), jnp.float32)
    acc_r = jnp.zeros((8, 128), jnp.float32)
    for ch in range(6):
        for i in range(bbr // 8):
            p = p_ref[ch, i * 8:(i + 1) * 8, :]
            t = t_ref[ch, i * 8:(i + 1) * 8, :]
            if ch < 3:
                d = p - t
                acc_t = acc_t + d * d
            else:
                d = (p - t) + (_wrap_correction(p) - _wrap_correction(t))
                acc_r = acc_r + d * d

    @pl.when(j == 0)
    def _():
        acc_ref[0:8, :] = acc_t
        acc_ref[8:16, :] = acc_r

    @pl.when(j > 0)
    def _():
        acc_ref[0:8, :] += acc_t
        acc_ref[8:16, :] += acc_r

    @pl.when(j == n_steps - 1)
    def _():
        s_trans = jnp.sum(acc_ref[0:8, :])
        s_rot = jnp.sum(acc_ref[8:16, :])
        trans_loss = s_trans * inv_n * _TRANS_WEIGHT
        rot_loss = s_rot * inv_n * _ROT_WEIGHT
        out_ref[0, 0] = trans_loss + rot_loss
        out_ref[0, 1] = trans_loss
        out_ref[0, 2] = rot_loss


def kernel(pred, target, *, interpret=False):
    B, T, D = pred.shape
    p3 = jnp.transpose(pred, (2, 0, 1))
    t3 = jnp.transpose(target, (2, 0, 1))

    BBR = 1024
    G = B // BBR
    n_per_half = B * T * 3
    out = pl.pallas_call(
        functools.partial(
            _loss_kernel, n_steps=G, bbr=BBR, inv_n=np.float32(1.0 / n_per_half)
        ),
        grid=(G,),
        in_specs=[
            pl.BlockSpec((D, BBR, T), lambda j: (0, j, 0)),
            pl.BlockSpec((D, BBR, T), lambda j: (0, j, 0)),
        ],
        out_specs=pl.BlockSpec(memory_space=pltpu.SMEM),
        out_shape=jax.ShapeDtypeStruct((1, 3), jnp.float32),
        scratch_shapes=[pltpu.VMEM((16, 128), jnp.float32)],
        compiler_params=pltpu.CompilerParams(
            dimension_semantics=("arbitrary",),
        ),
        name="weighted_mse_loss",
        interpret=interpret,
    )(p3, t3)

    return (out[0, 0], out[0, 1], out[0, 2])
